# baseline (speedup 1.0000x reference)
import functools

import jax
import jax.numpy as jnp
from jax import lax
from jax.experimental import pallas as pl
from jax.experimental.pallas import tpu as pltpu
from jax.experimental.pallas import tpu_sc as plsc

NC = 2
NS = 16
EK = 128
PAD = 1024


def _spmm_sc(g, z, src, dst, do_gather=True):
    N, D = g.shape
    CPT = src.shape[1]

    G = N // 8
    base_g = G // NS
    rem = G - base_g * NS

    def _row_ranges(sid, fn):
        if rem:
            @pl.when(sid < rem)
            def _():
                fn(sid * (base_g + 1) * 8, (base_g + 1) * 8)

            @pl.when(sid >= rem)
            def _():
                fn((rem * (base_g + 1) + (sid - rem) * base_g) * 8, base_g * 8)
        else:
            fn(sid * base_g * 8, base_g * 8)

    mesh = plsc.VectorSubcoreMesh(core_axis_name="c", subcore_axis_name="s")

    if not do_gather:
        @functools.partial(
            pl.kernel,
            mesh=mesh,
            out_type=jax.ShapeDtypeStruct((NC, N, D), jnp.float32),
            scratch_types=[
                pltpu.VMEM((CPT, EK), jnp.int32),
                pltpu.VMEM((EK, D), jnp.float32),
                pltpu.VMEM_SHARED((N + PAD, D), jnp.float32),
                pltpu.SemaphoreType.DMA,
            ],
        )
        def kd(g_hbm, z_hbm, src_hbm, dst_hbm, out_hbm, dst_v, rows0, acc,
               sem):
            cid = lax.axis_index("c")
            sid = lax.axis_index("s")
            wid = cid * NS + sid
            pltpu.async_copy(dst_hbm.at[wid], dst_v, sem)

            @pl.when(cid == 0)
            def _():
                _row_ranges(sid, lambda off, cnt: pltpu.sync_copy(
                    g_hbm.at[pl.ds(off, cnt)], acc.at[pl.ds(off, cnt)]))

            @pl.when(cid == 1)
            def _():
                _row_ranges(sid, lambda off, cnt: pltpu.sync_copy(
                    z_hbm.at[pl.ds(off, cnt)], acc.at[pl.ds(off, cnt)]))

            def fill(r, carry):
                for j in range(D // 16):
                    rows0[r, pl.ds(j * 16, 16)] = jnp.full((16,), 1.0,
                                                           jnp.float32)
                return carry
            lax.fori_loop(0, EK, fill, 0)
            pltpu.make_async_copy(dst_hbm.at[wid], dst_v, sem).wait()
            plsc.subcore_barrier()

            def fire(i, carry):
                pltpu.async_copy(rows0, acc.at[dst_v.at[i]], sem, add=True)
                return carry

            lax.fori_loop(0, CPT, fire, 0)

            def drain(i, carry):
                pltpu.make_async_copy(rows0, acc.at[dst_v.at[0]], sem).wait()
                return carry

            lax.fori_loop(0, CPT, drain, 0)
            plsc.subcore_barrier()
            _row_ranges(sid, lambda off, cnt: pltpu.sync_copy(
                acc.at[pl.ds(off, cnt)], out_hbm.at[cid, pl.ds(off, cnt)]))

        return kd(g, z, src, dst)

    NP = CPT // 2
    NQ = NP // 2
    srcq = src.reshape(NC * NS, NP, 2, EK)
    dstq = dst.reshape(NC * NS, NP, 2, EK)

    @functools.partial(
        pl.kernel,
        mesh=mesh,
        out_type=jax.ShapeDtypeStruct((NC, N, D), jnp.float32),
        scratch_types=[
            pltpu.VMEM((2, EK), jnp.int32),
            pltpu.VMEM((2, EK), jnp.int32),
            pltpu.VMEM((2, EK), jnp.int32),
            pltpu.VMEM((2, EK), jnp.int32),
            pltpu.VMEM((EK, D), jnp.float32),
            pltpu.VMEM((EK, D), jnp.float32),
            pltpu.VMEM_SHARED((N + PAD, D), jnp.float32),
            pltpu.SemaphoreType.DMA,
            pltpu.SemaphoreType.DMA,
            pltpu.SemaphoreType.DMA,
        ],
    )
    def k(g_hbm, z_hbm, srcq_hbm, dstq_hbm, out_hbm, sA, dA, sB, dB, rows0,
          rows1, acc, gsem, isem, ssem):
        cid = lax.axis_index("c")
        sid = lax.axis_index("s")
        wid = cid * NS + sid

        def iload(p, s, d):
            pltpu.async_copy(srcq_hbm.at[wid, p], s, isem)
            pltpu.async_copy(dstq_hbm.at[wid, p], d, isem)

        def iwait(p, s, d):
            pltpu.make_async_copy(srcq_hbm.at[wid, p], s, isem).wait()
            pltpu.make_async_copy(dstq_hbm.at[wid, p], d, isem).wait()

        def gather(s, b, buf):
            pltpu.async_copy(g_hbm.at[s.at[b]], buf, gsem)

        def gwait(s, b, buf):
            pltpu.make_async_copy(g_hbm.at[s.at[b]], buf, gsem).wait()

        def scat(buf, d, b):
            pltpu.async_copy(buf, acc.at[d.at[b]], ssem, add=True)

        def swait():
            pltpu.make_async_copy(rows0, acc.at[dA.at[0]], ssem).wait()

        iload(0, sA, dA)
        @pl.when(cid == 0)
        def _():
            _row_ranges(sid, lambda off, cnt: pltpu.sync_copy(
                g_hbm.at[pl.ds(off, cnt)], acc.at[pl.ds(off, cnt)]))

        @pl.when(cid == 1)
        def _():
            _row_ranges(sid, lambda off, cnt: pltpu.sync_copy(
                z_hbm.at[pl.ds(off, cnt)], acc.at[pl.ds(off, cnt)]))

        iwait(0, sA, dA)
        plsc.subcore_barrier()
        gather(sA, 0, rows0)
        iload(1, sB, dB)

        def quad(q, carry):
            pa = 2 * q
            pb = 2 * q + 1
            gwait(sA, 0, rows0)
            scat(rows0, dA, 0)

            @pl.when(q > 0)
            def _():
                swait()
                iload(pb, sB, dB)

            gather(sA, 1, rows1)
            gwait(sA, 1, rows1)
            scat(rows1, dA, 1)
            swait()
            iwait(pb, sB, dB)
            gather(sB, 0, rows0)
            gwait(sB, 0, rows0)
            scat(rows0, dB, 0)
            swait()

            @pl.when(q < NQ - 1)
            def _():
                iload(pa + 2, sA, dA)

            gather(sB, 1, rows1)
            gwait(sB, 1, rows1)
            scat(rows1, dB, 1)
            swait()

            @pl.when(q < NQ - 1)
            def _():
                iwait(pa + 2, sA, dA)
                gather(sA, 0, rows0)

            return carry

        lax.fori_loop(0, NQ, quad, 0)
        swait()
        plsc.subcore_barrier()
        _row_ranges(sid, lambda off, cnt: pltpu.sync_copy(
            acc.at[pl.ds(off, cnt)], out_hbm.at[cid, pl.ds(off, cnt)]))

    return k(g, z, srcq, dstq)


_BN = 2000


def _tc_matmul(x, W0):
    N, D = x.shape
    H = W0.shape[1]

    def body(x_ref, w_ref, hw_ref):
        hw_ref[...] = jnp.dot(x_ref[...], w_ref[...],
                              preferred_element_type=jnp.float32)

    return pl.pallas_call(
        body,
        grid=(N // _BN,),
        in_specs=[
            pl.BlockSpec((_BN, D), lambda i: (i, 0)),
            pl.BlockSpec((D, H), lambda i: (0, 0)),
        ],
        out_specs=pl.BlockSpec((_BN, H), lambda i: (i, 0)),
        out_shape=jax.ShapeDtypeStruct((N, H), jnp.float32),
    )(x, W0)


def _tc_first(deg_p, hw):
    N, H = hw.shape

    def body(dp_ref, hw_ref, g_ref, dinvc_ref):
        deg = dp_ref[0, :, 0:1] + dp_ref[1, :, 0:1]
        dinv = lax.rsqrt(deg)
        g_ref[...] = hw_ref[...] * dinv
        dinvc_ref[...] = dinv

    return pl.pallas_call(
        body,
        grid=(N // _BN,),
        in_specs=[
            pl.BlockSpec((NC, _BN, 16), lambda i: (0, i, 0)),
            pl.BlockSpec((_BN, H), lambda i: (i, 0)),
        ],
        out_specs=[
            pl.BlockSpec((_BN, H), lambda i: (i, 0)),
            pl.BlockSpec((_BN, 1), lambda i: (i, 0)),
        ],
        out_shape=[
            jax.ShapeDtypeStruct((N, H), jnp.float32),
            jax.ShapeDtypeStruct((N, 1), jnp.float32),
        ],
    )(deg_p, hw)


def _tc_mid(sp, dinvc, b, Wn):
    NCp, N, H = sp.shape

    def body(sp_ref, dinvc_ref, b_ref, w_ref, h_ref, g_ref):
        dinv = dinvc_ref[...]
        s = sp_ref[0] + sp_ref[1]
        h = jnp.maximum(dinv * s + b_ref[...], 0.0)
        h_ref[...] = h
        g_ref[...] = jnp.dot(h, w_ref[...], preferred_element_type=jnp.float32) * dinv

    return pl.pallas_call(
        body,
        grid=(N // _BN,),
        in_specs=[
            pl.BlockSpec((NC, _BN, H), lambda i: (0, i, 0)),
            pl.BlockSpec((_BN, 1), lambda i: (i, 0)),
            pl.BlockSpec((1, H), lambda i: (0, 0)),
            pl.BlockSpec((H, H), lambda i: (0, 0)),
        ],
        out_specs=[
            pl.BlockSpec((_BN, H), lambda i: (i, 0)),
            pl.BlockSpec((_BN, H), lambda i: (i, 0)),
        ],
        out_shape=[
            jax.ShapeDtypeStruct((N, H), jnp.float32),
            jax.ShapeDtypeStruct((N, H), jnp.float32),
        ],
    )(sp, dinvc, b, Wn)


def _tc_last(sp, dinvc, b, h1, h2, Wjk, bjk):
    NCp, N, H = sp.shape
    O = Wjk.shape[1]

    def body(sp_ref, dinvc_ref, b_ref, h1_ref, h2_ref, wjk_ref,
             bjk_ref, out_ref):
        dinv = dinvc_ref[...]
        s = sp_ref[0] + sp_ref[1]
        h3 = jnp.maximum(dinv * s + b_ref[...], 0.0)
        jk = (h1_ref[...] + h2_ref[...] + h3) * (1.0 / 3.0)
        out_ref[...] = (
            jnp.dot(jk, wjk_ref[...], preferred_element_type=jnp.float32)
            + bjk_ref[...]
        )

    return pl.pallas_call(
        body,
        grid=(N // _BN,),
        in_specs=[
            pl.BlockSpec((NC, _BN, H), lambda i: (0, i, 0)),
            pl.BlockSpec((_BN, 1), lambda i: (i, 0)),
            pl.BlockSpec((1, H), lambda i: (0, 0)),
            pl.BlockSpec((_BN, H), lambda i: (i, 0)),
            pl.BlockSpec((_BN, H), lambda i: (i, 0)),
            pl.BlockSpec((H, O), lambda i: (0, 0)),
            pl.BlockSpec((1, O), lambda i: (0, 0)),
        ],
        out_specs=pl.BlockSpec((_BN, O), lambda i: (i, 0)),
        out_shape=jax.ShapeDtypeStruct((N, O), jnp.float32),
    )(sp, dinvc, b, h1, h2, Wjk, bjk)


def kernel(x, edge_index, W0, b0, W1, b1, W2, b2, Wjk, bjk):
    N = x.shape[0]
    E = edge_index.shape[1]
    ei = edge_index.astype(jnp.int32)

    CPT = 2 * pl.cdiv(E, 2 * NC * NS * EK)
    Ep = NC * NS * CPT * EK
    src = jnp.concatenate(
        [ei[0], jnp.arange(Ep - E, dtype=jnp.int32) % N])
    dst = jnp.concatenate(
        [ei[1], N + (jnp.arange(Ep - E, dtype=jnp.int32) % PAD)])
    src = src.reshape(NC * NS, CPT, EK)
    dst = dst.reshape(NC * NS, CPT, EK)

    ones16 = jnp.ones((N, 16), jnp.float32)
    z16 = jnp.zeros((N, 16), jnp.float32)
    z = jnp.zeros((N, W0.shape[1]), jnp.float32)
    hw0 = _tc_matmul(x, W0)
    deg_p = _spmm_sc(ones16, z16, src, dst, do_gather=False)

    g0, dinvc = _tc_first(deg_p, hw0)
    sp0 = _spmm_sc(g0, z, src, dst)
    h1, g1 = _tc_mid(sp0, dinvc, b0.reshape(1, -1), W1)
    sp1 = _spmm_sc(g1, z, src, dst)
    h2, g2 = _tc_mid(sp1, dinvc, b1.reshape(1, -1), W2)
    sp2 = _spmm_sc(g2, z, src, dst)
    return _tc_last(sp2, dinvc, b2.reshape(1, -1), h1, h2, Wjk,
                    bjk.reshape(1, -1))


# speedup vs baseline: 1.0014x; 1.0014x over previous
import functools

import jax
import jax.numpy as jnp
from jax import lax
from jax.experimental import pallas as pl
from jax.experimental.pallas import tpu as pltpu
from jax.experimental.pallas import tpu_sc as plsc

NC = 2
NS = 16
EK = 128
PAD = 1024


def _spmm_sc(g, z, src, dst, do_gather=True):
    N, D = g.shape
    CPT = src.shape[1]

    G = N // 8
    base_g = G // NS
    rem = G - base_g * NS

    def _row_ranges(sid, fn):
        if rem:
            @pl.when(sid < rem)
            def _():
                fn(sid * (base_g + 1) * 8, (base_g + 1) * 8)

            @pl.when(sid >= rem)
            def _():
                fn((rem * (base_g + 1) + (sid - rem) * base_g) * 8, base_g * 8)
        else:
            fn(sid * base_g * 8, base_g * 8)

    mesh = plsc.VectorSubcoreMesh(core_axis_name="c", subcore_axis_name="s")

    if not do_gather:
        @functools.partial(
            pl.kernel,
            mesh=mesh,
            out_type=jax.ShapeDtypeStruct((NC, N, D), jnp.float32),
            scratch_types=[
                pltpu.VMEM((CPT, EK), jnp.int32),
                pltpu.VMEM((EK, D), jnp.float32),
                pltpu.VMEM_SHARED((N + PAD, D), jnp.float32),
                pltpu.SemaphoreType.DMA,
            ],
        )
        def kd(g_hbm, z_hbm, src_hbm, dst_hbm, out_hbm, dst_v, rows0, acc,
               sem):
            cid = lax.axis_index("c")
            sid = lax.axis_index("s")
            wid = cid * NS + sid
            pltpu.async_copy(dst_hbm.at[wid], dst_v, sem)

            @pl.when(cid == 0)
            def _():
                _row_ranges(sid, lambda off, cnt: pltpu.sync_copy(
                    g_hbm.at[pl.ds(off, cnt)], acc.at[pl.ds(off, cnt)]))

            @pl.when(cid == 1)
            def _():
                _row_ranges(sid, lambda off, cnt: pltpu.sync_copy(
                    z_hbm.at[pl.ds(off, cnt)], acc.at[pl.ds(off, cnt)]))

            def fill(r, carry):
                for j in range(D // 16):
                    rows0[r, pl.ds(j * 16, 16)] = jnp.full((16,), 1.0,
                                                           jnp.float32)
                return carry
            lax.fori_loop(0, EK, fill, 0)
            pltpu.make_async_copy(dst_hbm.at[wid], dst_v, sem).wait()
            plsc.subcore_barrier()

            def fire(i, carry):
                pltpu.async_copy(rows0, acc.at[dst_v.at[i]], sem, add=True)
                return carry

            lax.fori_loop(0, CPT, fire, 0)

            def drain(i, carry):
                pltpu.make_async_copy(rows0, acc.at[dst_v.at[0]], sem).wait()
                return carry

            lax.fori_loop(0, CPT, drain, 0)
            plsc.subcore_barrier()
            _row_ranges(sid, lambda off, cnt: pltpu.sync_copy(
                acc.at[pl.ds(off, cnt)], out_hbm.at[cid, pl.ds(off, cnt)]))

        return kd(g, z, src, dst)

    NP = CPT // 2
    NQ = NP // 2
    srcq = src.reshape(NC * NS, NP, 2, EK)
    dstq = dst.reshape(NC * NS, NP, 2, EK)

    @functools.partial(
        pl.kernel,
        mesh=mesh,
        out_type=jax.ShapeDtypeStruct((NC, N, D), jnp.float32),
        scratch_types=[
            pltpu.VMEM((2, EK), jnp.int32),
            pltpu.VMEM((2, EK), jnp.int32),
            pltpu.VMEM((2, EK), jnp.int32),
            pltpu.VMEM((2, EK), jnp.int32),
            pltpu.VMEM((EK, D), jnp.float32),
            pltpu.VMEM((EK, D), jnp.float32),
            pltpu.VMEM_SHARED((N + PAD, D), jnp.float32),
            pltpu.SemaphoreType.DMA,
            pltpu.SemaphoreType.DMA,
            pltpu.SemaphoreType.DMA,
        ],
    )
    def k(g_hbm, z_hbm, srcq_hbm, dstq_hbm, out_hbm, sA, dA, sB, dB, rows0,
          rows1, acc, gsem, isem, ssem):
        cid = lax.axis_index("c")
        sid = lax.axis_index("s")
        wid = cid * NS + sid

        def iload(p, s, d):
            pltpu.async_copy(srcq_hbm.at[wid, p], s, isem)
            pltpu.async_copy(dstq_hbm.at[wid, p], d, isem)

        def iwait(p, s, d):
            pltpu.make_async_copy(srcq_hbm.at[wid, p], s, isem).wait()
            pltpu.make_async_copy(dstq_hbm.at[wid, p], d, isem).wait()

        def gather(s, b, buf):
            pltpu.async_copy(g_hbm.at[s.at[b]], buf, gsem)

        def gwait(s, b, buf):
            pltpu.make_async_copy(g_hbm.at[s.at[b]], buf, gsem).wait()

        def scat(buf, d, b):
            pltpu.async_copy(buf, acc.at[d.at[b]], ssem, add=True)

        def swait():
            pltpu.make_async_copy(rows0, acc.at[dA.at[0]], ssem).wait()

        iload(0, sA, dA)
        @pl.when(cid == 0)
        def _():
            _row_ranges(sid, lambda off, cnt: pltpu.sync_copy(
                g_hbm.at[pl.ds(off, cnt)], acc.at[pl.ds(off, cnt)]))

        @pl.when(cid == 1)
        def _():
            _row_ranges(sid, lambda off, cnt: pltpu.sync_copy(
                z_hbm.at[pl.ds(off, cnt)], acc.at[pl.ds(off, cnt)]))

        iwait(0, sA, dA)
        plsc.subcore_barrier()
        gather(sA, 0, rows0)
        iload(1, sB, dB)

        def quad(q, carry):
            pa = 2 * q
            pb = 2 * q + 1
            gwait(sA, 0, rows0)
            scat(rows0, dA, 0)

            @pl.when(q > 0)
            def _():
                swait()
                iload(pb, sB, dB)

            gather(sA, 1, rows1)
            gwait(sA, 1, rows1)
            scat(rows1, dA, 1)
            swait()
            iwait(pb, sB, dB)
            gather(sB, 0, rows0)
            gwait(sB, 0, rows0)
            scat(rows0, dB, 0)
            swait()

            @pl.when(q < NQ - 1)
            def _():
                iload(pa + 2, sA, dA)

            gather(sB, 1, rows1)
            gwait(sB, 1, rows1)
            scat(rows1, dB, 1)
            swait()

            @pl.when(q < NQ - 1)
            def _():
                iwait(pa + 2, sA, dA)
                gather(sA, 0, rows0)

            return carry

        lax.fori_loop(0, NQ, quad, 0)
        swait()
        plsc.subcore_barrier()
        _row_ranges(sid, lambda off, cnt: pltpu.sync_copy(
            acc.at[pl.ds(off, cnt)], out_hbm.at[cid, pl.ds(off, cnt)]))

    return k(g, z, srcq, dstq)


_BN = 2000


def _tc_first(deg_p, x, W0):
    N, D = x.shape
    H = W0.shape[1]

    def body(dp_ref, x_ref, w_ref, g_ref, dinvc_ref):
        deg = dp_ref[0, :, 0:1] + dp_ref[1, :, 0:1]
        dinv = lax.rsqrt(deg)
        hw = jnp.dot(x_ref[...], w_ref[...], preferred_element_type=jnp.float32)
        g_ref[...] = hw * dinv
        dinvc_ref[...] = dinv

    return pl.pallas_call(
        body,
        grid=(N // _BN,),
        in_specs=[
            pl.BlockSpec((NC, _BN, 16), lambda i: (0, i, 0)),
            pl.BlockSpec((_BN, D), lambda i: (i, 0)),
            pl.BlockSpec((D, H), lambda i: (0, 0)),
        ],
        out_specs=[
            pl.BlockSpec((_BN, H), lambda i: (i, 0)),
            pl.BlockSpec((_BN, 1), lambda i: (i, 0)),
        ],
        out_shape=[
            jax.ShapeDtypeStruct((N, H), jnp.float32),
            jax.ShapeDtypeStruct((N, 1), jnp.float32),
        ],
    )(deg_p, x, W0)


def _tc_mid(sp, dinvc, b, Wn):
    NCp, N, H = sp.shape

    def body(sp_ref, dinvc_ref, b_ref, w_ref, h_ref, g_ref):
        dinv = dinvc_ref[...]
        s = sp_ref[0] + sp_ref[1]
        h = jnp.maximum(dinv * s + b_ref[...], 0.0)
        h_ref[...] = h
        g_ref[...] = jnp.dot(h, w_ref[...], preferred_element_type=jnp.float32) * dinv

    return pl.pallas_call(
        body,
        grid=(N // _BN,),
        in_specs=[
            pl.BlockSpec((NC, _BN, H), lambda i: (0, i, 0)),
            pl.BlockSpec((_BN, 1), lambda i: (i, 0)),
            pl.BlockSpec((1, H), lambda i: (0, 0)),
            pl.BlockSpec((H, H), lambda i: (0, 0)),
        ],
        out_specs=[
            pl.BlockSpec((_BN, H), lambda i: (i, 0)),
            pl.BlockSpec((_BN, H), lambda i: (i, 0)),
        ],
        out_shape=[
            jax.ShapeDtypeStruct((N, H), jnp.float32),
            jax.ShapeDtypeStruct((N, H), jnp.float32),
        ],
    )(sp, dinvc, b, Wn)


def _tc_last(sp, dinvc, b, h1, h2, Wjk, bjk):
    NCp, N, H = sp.shape
    O = Wjk.shape[1]

    def body(sp_ref, dinvc_ref, b_ref, h1_ref, h2_ref, wjk_ref,
             bjk_ref, out_ref):
        dinv = dinvc_ref[...]
        s = sp_ref[0] + sp_ref[1]
        h3 = jnp.maximum(dinv * s + b_ref[...], 0.0)
        jk = (h1_ref[...] + h2_ref[...] + h3) * (1.0 / 3.0)
        out_ref[...] = (
            jnp.dot(jk, wjk_ref[...], preferred_element_type=jnp.float32)
            + bjk_ref[...]
        )

    return pl.pallas_call(
        body,
        grid=(N // _BN,),
        in_specs=[
            pl.BlockSpec((NC, _BN, H), lambda i: (0, i, 0)),
            pl.BlockSpec((_BN, 1), lambda i: (i, 0)),
            pl.BlockSpec((1, H), lambda i: (0, 0)),
            pl.BlockSpec((_BN, H), lambda i: (i, 0)),
            pl.BlockSpec((_BN, H), lambda i: (i, 0)),
            pl.BlockSpec((H, O), lambda i: (0, 0)),
            pl.BlockSpec((1, O), lambda i: (0, 0)),
        ],
        out_specs=pl.BlockSpec((_BN, O), lambda i: (i, 0)),
        out_shape=jax.ShapeDtypeStruct((N, O), jnp.float32),
    )(sp, dinvc, b, h1, h2, Wjk, bjk)


def kernel(x, edge_index, W0, b0, W1, b1, W2, b2, Wjk, bjk):
    N = x.shape[0]
    E = edge_index.shape[1]
    ei = edge_index.astype(jnp.int32)

    CPT = 2 * pl.cdiv(E, 2 * NC * NS * EK)
    Ep = NC * NS * CPT * EK
    src = jnp.concatenate(
        [ei[0], jnp.arange(Ep - E, dtype=jnp.int32) % N])
    dst = jnp.concatenate(
        [ei[1], N + (jnp.arange(Ep - E, dtype=jnp.int32) % PAD)])
    src = src.reshape(NC * NS, CPT, EK)
    dst = dst.reshape(NC * NS, CPT, EK)

    ones16 = jnp.ones((N, 16), jnp.float32)
    z16 = jnp.zeros((N, 16), jnp.float32)
    z = jnp.zeros((N, W0.shape[1]), jnp.float32)
    deg_p = _spmm_sc(ones16, z16, src, dst, do_gather=False)

    g0, dinvc = _tc_first(deg_p, x, W0)
    sp0 = _spmm_sc(g0, z, src, dst)
    h1, g1 = _tc_mid(sp0, dinvc, b0.reshape(1, -1), W1)
    sp1 = _spmm_sc(g1, z, src, dst)
    h2, g2 = _tc_mid(sp1, dinvc, b1.reshape(1, -1), W2)
    sp2 = _spmm_sc(g2, z, src, dst)
    return _tc_last(sp2, dinvc, b2.reshape(1, -1), h1, h2, Wjk,
                    bjk.reshape(1, -1))
